# baseline (speedup 1.0000x reference)
import functools

import jax
import jax.numpy as jnp
from jax import lax
from jax.experimental import pallas as pl
from jax.experimental.pallas import tpu as pltpu
from jax.experimental.pallas import tpu_sc as plsc

N_NODES = 10000
N_EDGES = 160000
D = 256
NUM_CORES = 2
NUM_SUBCORES = 16
NUM_TILES = NUM_CORES * NUM_SUBCORES
EDGES_PER_TILE = 5200
E_PAD = NUM_TILES * EDGES_PER_TILE
CHUNK = 80
SUPER = 1040
NUM_SUPER = EDGES_PER_TILE // SUPER
CHUNKS_PER_SUPER = SUPER // CHUNK
N_ROWS = N_NODES + 16
TRASH = N_NODES
ROWS_PER_TILE = N_ROWS // NUM_SUBCORES
DEG_W = 16


def _sc_segment_sum(xb, src, dst, zrow, zdeg, ones):
  mesh = plsc.VectorSubcoreMesh(core_axis_name="c", subcore_axis_name="s")

  @functools.partial(
      pl.kernel,
      mesh=mesh,
      compiler_params=pltpu.CompilerParams(use_tc_tiling_on_sc=False),
      out_type=[
          jax.ShapeDtypeStruct((NUM_CORES, N_ROWS, D), jnp.bfloat16),
          jax.ShapeDtypeStruct((NUM_CORES, N_ROWS, DEG_W), jnp.float32),
      ],
      scratch_types=[
          pltpu.VMEM_SHARED((N_ROWS, D), jnp.bfloat16),
          pltpu.VMEM_SHARED((N_ROWS, DEG_W), jnp.float32),
          pltpu.VMEM((SUPER,), jnp.int32),
          pltpu.VMEM((SUPER,), jnp.int32),
          pltpu.VMEM((CHUNK, D), jnp.bfloat16),
          pltpu.VMEM((CHUNK, D), jnp.bfloat16),
          pltpu.VMEM((CHUNK,), jnp.int32),
          pltpu.VMEM((CHUNK,), jnp.int32),
          pltpu.VMEM((CHUNK, DEG_W), jnp.float32),
          pltpu.SemaphoreType.DMA,
          pltpu.SemaphoreType.DMA,
          pltpu.SemaphoreType.DMA,
          pltpu.SemaphoreType.DMA,
          pltpu.SemaphoreType.DMA,
          pltpu.SemaphoreType.DMA,
      ],
  )
  def body(xb_hbm, src_hbm, dst_hbm, zrow_hbm, zdeg_hbm, ones_hbm,
           agg_out, deg_out, agg_sh, deg_sh, src_v, dst_v, rows_a, rows_b,
           dsti_a, dsti_b, ones_v, gsem_a, gsem_b, ssem_a, ssem_b, dsem_a,
           dsem_b):
    c = lax.axis_index("c")
    s = lax.axis_index("s")
    e0 = (c * NUM_SUBCORES + s) * EDGES_PER_TILE
    r0 = s * ROWS_PER_TILE

    pltpu.sync_copy(zrow_hbm, agg_sh.at[pl.ds(r0, ROWS_PER_TILE)])
    pltpu.sync_copy(zdeg_hbm, deg_sh.at[pl.ds(r0, ROWS_PER_TILE)])
    pltpu.sync_copy(ones_hbm, ones_v)

    plsc.subcore_barrier()

    def gather_start(i, rows_buf, sem):
      pltpu.async_copy(
          xb_hbm.at[src_v.at[pl.ds(i * CHUNK, CHUNK)]], rows_buf, sem)

    def gather_drain(rows_buf, sem):
      pltpu.make_async_copy(xb_hbm.at[pl.ds(0, CHUNK)], rows_buf, sem).wait()

    def stage(i, dsti_buf):
      for j in range(CHUNK // 16):
        dsti_buf[pl.ds(j * 16, 16)] = dst_v[pl.ds(i * CHUNK + j * 16, 16)]

    def scatter_start(rows_buf, dsti_buf, ssem, dsem):
      pltpu.async_copy(rows_buf, agg_sh.at[dsti_buf], ssem, add=True)
      pltpu.async_copy(ones_v, deg_sh.at[dsti_buf], dsem, add=True)

    def scatter_drain(rows_buf, ssem, dsem):
      pltpu.make_async_copy(rows_buf, agg_sh.at[pl.ds(0, CHUNK)], ssem).wait()
      pltpu.make_async_copy(ones_v, deg_sh.at[pl.ds(0, CHUNK)], dsem).wait()

    def super_body(sc, _):
      pltpu.sync_copy(src_hbm.at[pl.ds(e0 + sc * SUPER, SUPER)], src_v)
      pltpu.sync_copy(dst_hbm.at[pl.ds(e0 + sc * SUPER, SUPER)], dst_v)

      gather_start(0, rows_a, gsem_a)

      def pair_body(p, _):
        i0 = 2 * p
        gather_drain(rows_a, gsem_a)
        stage(i0, dsti_a)
        gather_start(i0 + 1, rows_b, gsem_b)
        scatter_start(rows_a, dsti_a, ssem_a, dsem_a)
        gather_drain(rows_b, gsem_b)
        stage(i0 + 1, dsti_b)
        scatter_drain(rows_a, ssem_a, dsem_a)
        scatter_start(rows_b, dsti_b, ssem_b, dsem_b)
        gather_start(i0 + 2, rows_a, gsem_a)
        scatter_drain(rows_b, ssem_b, dsem_b)
        return 0

      lax.fori_loop(0, (CHUNKS_PER_SUPER - 1) // 2, pair_body, 0)

      gather_drain(rows_a, gsem_a)
      stage(CHUNKS_PER_SUPER - 1, dsti_a)
      scatter_start(rows_a, dsti_a, ssem_a, dsem_a)
      scatter_drain(rows_a, ssem_a, dsem_a)
      return 0

    lax.fori_loop(0, NUM_SUPER, super_body, 0)

    plsc.subcore_barrier()

    pltpu.sync_copy(agg_sh.at[pl.ds(r0, ROWS_PER_TILE)],
                    agg_out.at[c, pl.ds(r0, ROWS_PER_TILE)])
    pltpu.sync_copy(deg_sh.at[pl.ds(r0, ROWS_PER_TILE)],
                    deg_out.at[c, pl.ds(r0, ROWS_PER_TILE)])

  return body(xb, src, dst, zrow, zdeg, ones)


ROW_BLOCK = 1000
GRID = N_NODES // ROW_BLOCK


def _tc_body(x_ref, agg0_ref, agg1_ref, deg0_ref, deg1_ref, ws_ref, wn_ref,
             b_ref, gamma_ref, beta_ref, out_ref):
  deg = deg0_ref[:, 0:1] + deg1_ref[:, 0:1]
  scale = 1.0 / jnp.maximum(deg, 1.0)
  agg = agg0_ref[:].astype(jnp.float32) + agg1_ref[:].astype(jnp.float32)
  h = jnp.dot(x_ref[:], ws_ref[:], preferred_element_type=jnp.float32)
  h += jnp.dot(agg * scale, wn_ref[:], preferred_element_type=jnp.float32)
  h += b_ref[:]
  mu = jnp.mean(h, axis=-1, keepdims=True)
  d = h - mu
  var = jnp.mean(d * d, axis=-1, keepdims=True)
  y = d * lax.rsqrt(var + 1e-5) * gamma_ref[:] + beta_ref[:]
  out_ref[:] = jnp.maximum(y, 0.0)


def _tc_dense(x, agg0, agg1, deg0, deg1, W_self, W_nbr, b, gamma, beta):
  row_spec = lambda w: pl.BlockSpec((ROW_BLOCK, w), lambda i: (i, 0))
  full_spec = lambda a, b_: pl.BlockSpec((a, b_), lambda i: (0, 0))
  return pl.pallas_call(
      _tc_body,
      grid=(GRID,),
      in_specs=[
          row_spec(D),
          row_spec(D),
          row_spec(D),
          row_spec(DEG_W),
          row_spec(DEG_W),
          full_spec(D, D),
          full_spec(D, D),
          full_spec(1, D),
          full_spec(1, D),
          full_spec(1, D),
      ],
      out_specs=row_spec(D),
      out_shape=jax.ShapeDtypeStruct((N_NODES, D), jnp.float32),
  )(x, agg0, agg1, deg0, deg1, W_self, W_nbr, b, gamma, beta)


def kernel(x, edge_index, W_self, W_nbr, b, gamma, beta):
  n_pad = E_PAD - N_EDGES
  src = jnp.concatenate(
      [edge_index[0].astype(jnp.int32), jnp.zeros((n_pad,), jnp.int32)])
  dst = jnp.concatenate(
      [edge_index[1].astype(jnp.int32), jnp.full((n_pad,), TRASH, jnp.int32)])
  xb = x.astype(jnp.bfloat16)
  zrow = jnp.zeros((ROWS_PER_TILE, D), jnp.bfloat16)
  zdeg = jnp.zeros((ROWS_PER_TILE, DEG_W), jnp.float32)
  ones = jnp.ones((CHUNK, DEG_W), jnp.float32)

  agg2, deg2 = _sc_segment_sum(xb, src, dst, zrow, zdeg, ones)

  return _tc_dense(x, agg2[0, :N_NODES], agg2[1, :N_NODES],
                   deg2[0, :N_NODES], deg2[1, :N_NODES],
                   W_self, W_nbr,
                   b.reshape(1, D), gamma.reshape(1, D), beta.reshape(1, D))


# speedup vs baseline: 1.1179x; 1.1179x over previous
import functools

import jax
import jax.numpy as jnp
from jax import lax
from jax.experimental import pallas as pl
from jax.experimental.pallas import tpu as pltpu
from jax.experimental.pallas import tpu_sc as plsc

N_NODES = 10000
N_EDGES = 160000
D = 256
NUM_CORES = 2
NUM_SUBCORES = 16
NUM_TILES = NUM_CORES * NUM_SUBCORES
EDGES_PER_TILE = 5200
E_PAD = NUM_TILES * EDGES_PER_TILE
CHUNK = 80
SUPER = 1040
NUM_SUPER = EDGES_PER_TILE // SUPER
CHUNKS_PER_SUPER = SUPER // CHUNK
N_ROWS = N_NODES + 16
TRASH = N_NODES
ROWS_PER_TILE = N_ROWS // NUM_SUBCORES
DEG_W = 16


def _sc_segment_sum(xb, src, dst, zrow, zdeg, ones):
  mesh = plsc.VectorSubcoreMesh(core_axis_name="c", subcore_axis_name="s")

  @functools.partial(
      pl.kernel,
      mesh=mesh,
      compiler_params=pltpu.CompilerParams(use_tc_tiling_on_sc=False),
      out_type=[
          jax.ShapeDtypeStruct((NUM_CORES, N_ROWS, D), jnp.bfloat16),
          jax.ShapeDtypeStruct((NUM_CORES, N_ROWS, DEG_W), jnp.float32),
      ],
      scratch_types=[
          pltpu.VMEM_SHARED((N_ROWS, D), jnp.bfloat16),
          pltpu.VMEM_SHARED((N_ROWS, DEG_W), jnp.float32),
          pltpu.VMEM((SUPER,), jnp.int32),
          pltpu.VMEM((SUPER,), jnp.int32),
          pltpu.VMEM((CHUNK, D), jnp.bfloat16),
          pltpu.VMEM((CHUNK, D), jnp.bfloat16),
          pltpu.VMEM((CHUNK,), jnp.int32),
          pltpu.VMEM((CHUNK,), jnp.int32),
          pltpu.VMEM((CHUNK, DEG_W), jnp.float32),
          pltpu.SemaphoreType.DMA,
          pltpu.SemaphoreType.DMA,
          pltpu.SemaphoreType.DMA,
          pltpu.SemaphoreType.DMA,
          pltpu.SemaphoreType.DMA,
          pltpu.SemaphoreType.DMA,
      ],
  )
  def body(xb_hbm, src_hbm, dst_hbm, zrow_hbm, zdeg_hbm, ones_hbm,
           agg_out, deg_out, agg_sh, deg_sh, src_v, dst_v, rows_a, rows_b,
           dsti_a, dsti_b, ones_v, gsem_a, gsem_b, ssem_a, ssem_b, dsem_a,
           dsem_b):
    c = lax.axis_index("c")
    s = lax.axis_index("s")
    e0 = (c * NUM_SUBCORES + s) * EDGES_PER_TILE
    r0 = s * ROWS_PER_TILE

    pltpu.sync_copy(zrow_hbm, agg_sh.at[pl.ds(r0, ROWS_PER_TILE)])
    pltpu.sync_copy(zdeg_hbm, deg_sh.at[pl.ds(r0, ROWS_PER_TILE)])
    pltpu.sync_copy(ones_hbm, ones_v)

    plsc.subcore_barrier()

    def gather_start(i, rows_buf, sem):
      pltpu.async_copy(
          xb_hbm.at[src_v.at[pl.ds(i * CHUNK, CHUNK)]], rows_buf, sem)

    def gather_drain(rows_buf, sem):
      pltpu.make_async_copy(xb_hbm.at[pl.ds(0, CHUNK)], rows_buf, sem).wait()

    def stage(i, dsti_buf):
      for j in range(CHUNK // 16):
        dsti_buf[pl.ds(j * 16, 16)] = dst_v[pl.ds(i * CHUNK + j * 16, 16)]

    def scatter_start(rows_buf, dsti_buf, ssem, dsem):
      pltpu.async_copy(rows_buf, agg_sh.at[dsti_buf], ssem, add=True)
      pltpu.async_copy(ones_v, deg_sh.at[dsti_buf], dsem, add=True)

    def scatter_drain(rows_buf, ssem, dsem):
      pltpu.make_async_copy(rows_buf, agg_sh.at[pl.ds(0, CHUNK)], ssem).wait()
      pltpu.make_async_copy(ones_v, deg_sh.at[pl.ds(0, CHUNK)], dsem).wait()

    def super_body(sc, _):
      pltpu.sync_copy(src_hbm.at[pl.ds(e0 + sc * SUPER, SUPER)], src_v)
      pltpu.sync_copy(dst_hbm.at[pl.ds(e0 + sc * SUPER, SUPER)], dst_v)

      gather_start(0, rows_a, gsem_a)

      def pair_body(p, _):
        i0 = 2 * p
        gather_drain(rows_a, gsem_a)
        stage(i0, dsti_a)
        gather_start(i0 + 1, rows_b, gsem_b)
        scatter_start(rows_a, dsti_a, ssem_a, dsem_a)
        gather_drain(rows_b, gsem_b)
        stage(i0 + 1, dsti_b)
        scatter_drain(rows_a, ssem_a, dsem_a)
        scatter_start(rows_b, dsti_b, ssem_b, dsem_b)
        gather_start(i0 + 2, rows_a, gsem_a)
        scatter_drain(rows_b, ssem_b, dsem_b)
        return 0

      lax.fori_loop(0, (CHUNKS_PER_SUPER - 1) // 2, pair_body, 0)

      gather_drain(rows_a, gsem_a)
      stage(CHUNKS_PER_SUPER - 1, dsti_a)
      scatter_start(rows_a, dsti_a, ssem_a, dsem_a)
      scatter_drain(rows_a, ssem_a, dsem_a)
      return 0

    lax.fori_loop(0, NUM_SUPER, super_body, 0)

    plsc.subcore_barrier()

    pltpu.sync_copy(agg_sh.at[pl.ds(r0, ROWS_PER_TILE)],
                    agg_out.at[c, pl.ds(r0, ROWS_PER_TILE)])
    pltpu.sync_copy(deg_sh.at[pl.ds(r0, ROWS_PER_TILE)],
                    deg_out.at[c, pl.ds(r0, ROWS_PER_TILE)])

  return body(xb, src, dst, zrow, zdeg, ones)


ROW_BLOCK = 1000
GRID = N_NODES // ROW_BLOCK


def _tc_body(x_ref, agg0_ref, agg1_ref, deg0_ref, deg1_ref, ws_ref, wn_ref,
             b_ref, gamma_ref, beta_ref, out_ref):
  deg = deg0_ref[:, 0:1] + deg1_ref[:, 0:1]
  scale = 1.0 / jnp.maximum(deg, 1.0)
  agg = agg0_ref[:].astype(jnp.float32) + agg1_ref[:].astype(jnp.float32)
  h = jnp.dot(x_ref[:], ws_ref[:], preferred_element_type=jnp.float32)
  h += jnp.dot(agg * scale, wn_ref[:], preferred_element_type=jnp.float32)
  h += b_ref[:]
  mu = jnp.mean(h, axis=-1, keepdims=True)
  d = h - mu
  var = jnp.mean(d * d, axis=-1, keepdims=True)
  y = d * lax.rsqrt(var + 1e-5) * gamma_ref[:] + beta_ref[:]
  out_ref[:] = jnp.maximum(y, 0.0)


def _tc_dense(x, agg0, agg1, deg0, deg1, W_self, W_nbr, b, gamma, beta):
  row_spec = lambda w: pl.BlockSpec((ROW_BLOCK, w), lambda i: (i, 0))
  full_spec = lambda a, b_: pl.BlockSpec((a, b_), lambda i: (0, 0))
  return pl.pallas_call(
      _tc_body,
      grid=(GRID,),
      in_specs=[
          row_spec(D),
          row_spec(D),
          row_spec(D),
          row_spec(DEG_W),
          row_spec(DEG_W),
          full_spec(D, D),
          full_spec(D, D),
          full_spec(1, D),
          full_spec(1, D),
          full_spec(1, D),
      ],
      out_specs=row_spec(D),
      out_shape=jax.ShapeDtypeStruct((N_NODES, D), jnp.float32),
  )(x, agg0, agg1, deg0, deg1, W_self, W_nbr, b, gamma, beta)


def kernel(x, edge_index, W_self, W_nbr, b, gamma, beta):
  real_per_tile = N_EDGES // NUM_TILES
  pad_per_tile = EDGES_PER_TILE - real_per_tile
  src = jnp.concatenate(
      [edge_index[0].astype(jnp.int32).reshape(NUM_TILES, real_per_tile),
       jnp.zeros((NUM_TILES, pad_per_tile), jnp.int32)], axis=1).reshape(-1)
  pad_dst = jnp.broadcast_to(
      TRASH + (jnp.arange(pad_per_tile, dtype=jnp.int32) % 16),
      (NUM_TILES, pad_per_tile))
  dst = jnp.concatenate(
      [edge_index[1].astype(jnp.int32).reshape(NUM_TILES, real_per_tile),
       pad_dst], axis=1).reshape(-1)
  xb = x.astype(jnp.bfloat16)
  zrow = jnp.zeros((ROWS_PER_TILE, D), jnp.bfloat16)
  zdeg = jnp.zeros((ROWS_PER_TILE, DEG_W), jnp.float32)
  ones = jnp.ones((CHUNK, DEG_W), jnp.float32)

  agg2, deg2 = _sc_segment_sum(xb, src, dst, zrow, zdeg, ones)

  return _tc_dense(x, agg2[0, :N_NODES], agg2[1, :N_NODES],
                   deg2[0, :N_NODES], deg2[1, :N_NODES],
                   W_self, W_nbr,
                   b.reshape(1, D), gamma.reshape(1, D), beta.reshape(1, D))


# speedup vs baseline: 2.2305x; 1.9952x over previous
import functools

import jax
import jax.numpy as jnp
from jax import lax
from jax.experimental import pallas as pl
from jax.experimental.pallas import tpu as pltpu
from jax.experimental.pallas import tpu_sc as plsc

N_NODES = 10000
N_EDGES = 160000
D = 256
DH = D // 2
NUM_CORES = 2
NUM_SUBCORES = 16
EDGES_PER_TILE = N_EDGES // NUM_SUBCORES
CHUNK = 80
SUPER = 2000
NUM_SUPER = EDGES_PER_TILE // SUPER
CHUNKS_PER_SUPER = SUPER // CHUNK
ROWS_PER_TILE = N_NODES // NUM_SUBCORES
DEG_W = 16


def _sc_segment_sum(xcat, src, dst, zrow, zdeg, ones):
  mesh = plsc.VectorSubcoreMesh(core_axis_name="c", subcore_axis_name="s")

  @functools.partial(
      pl.kernel,
      mesh=mesh,
      compiler_params=pltpu.CompilerParams(use_tc_tiling_on_sc=False),
      out_type=[
          jax.ShapeDtypeStruct((NUM_CORES, N_NODES, DH), jnp.float32),
          jax.ShapeDtypeStruct((N_NODES, DEG_W), jnp.float32),
      ],
      scratch_types=[
          pltpu.VMEM_SHARED((N_NODES, DH), jnp.float32),
          pltpu.VMEM_SHARED((N_NODES, DEG_W), jnp.float32),
          pltpu.VMEM((SUPER,), jnp.int32),
          pltpu.VMEM((SUPER,), jnp.int32),
          pltpu.VMEM((CHUNK, DH), jnp.float32),
          pltpu.VMEM((CHUNK, DH), jnp.float32),
          pltpu.VMEM((CHUNK,), jnp.int32),
          pltpu.VMEM((CHUNK,), jnp.int32),
          pltpu.VMEM((CHUNK, DEG_W), jnp.float32),
          pltpu.SemaphoreType.DMA,
          pltpu.SemaphoreType.DMA,
          pltpu.SemaphoreType.DMA,
          pltpu.SemaphoreType.DMA,
          pltpu.SemaphoreType.DMA,
          pltpu.SemaphoreType.DMA,
      ],
  )
  def body(xcat_hbm, src_hbm, dst_hbm, zrow_hbm, zdeg_hbm, ones_hbm,
           agg_out, deg_out, agg_sh, deg_sh, src_v, dst_v, rows_a, rows_b,
           dsti_a, dsti_b, ones_v, gsem_a, gsem_b, ssem_a, ssem_b, dsem_a,
           dsem_b):
    c = lax.axis_index("c")
    s = lax.axis_index("s")
    e0 = s * EDGES_PER_TILE
    r0 = s * ROWS_PER_TILE

    pltpu.sync_copy(zrow_hbm, agg_sh.at[pl.ds(r0, ROWS_PER_TILE)])

    @pl.when(c == 0)
    def _():
      pltpu.sync_copy(zdeg_hbm, deg_sh.at[pl.ds(r0, ROWS_PER_TILE)])
      pltpu.sync_copy(ones_hbm, ones_v)

    plsc.subcore_barrier()

    cvec = jnp.full((16,), c, dtype=jnp.int32)

    def gather_start(i, rows_buf, sem):
      pltpu.async_copy(
          xcat_hbm.at[src_v.at[pl.ds(i * CHUNK, CHUNK)]], rows_buf, sem)

    def gather_drain(rows_buf, sem):
      pltpu.make_async_copy(xcat_hbm.at[pl.ds(0, CHUNK)], rows_buf, sem).wait()

    def stage(i, dsti_buf):
      for j in range(CHUNK // 16):
        dsti_buf[pl.ds(j * 16, 16)] = dst_v[pl.ds(i * CHUNK + j * 16, 16)]

    def scatter_start(rows_buf, dsti_buf, ssem, dsem):
      pltpu.async_copy(rows_buf, agg_sh.at[dsti_buf], ssem, add=True)

      @pl.when(c == 0)
      def _():
        pltpu.async_copy(ones_v, deg_sh.at[dsti_buf], dsem, add=True)

    def scatter_drain(rows_buf, ssem, dsem):
      pltpu.make_async_copy(rows_buf, agg_sh.at[pl.ds(0, CHUNK)], ssem).wait()

      @pl.when(c == 0)
      def _():
        pltpu.make_async_copy(ones_v, deg_sh.at[pl.ds(0, CHUNK)], dsem).wait()

    def super_body(sc, _):
      pltpu.sync_copy(src_hbm.at[pl.ds(e0 + sc * SUPER, SUPER)], src_v)
      pltpu.sync_copy(dst_hbm.at[pl.ds(e0 + sc * SUPER, SUPER)], dst_v)

      def add_bias(i, _):
        src_v[pl.ds(i * 16, 16)] = src_v[pl.ds(i * 16, 16)] * 2 + cvec
        return 0

      lax.fori_loop(0, SUPER // 16, add_bias, 0)

      gather_start(0, rows_a, gsem_a)

      def pair_body(p, _):
        i0 = 2 * p
        gather_drain(rows_a, gsem_a)
        stage(i0, dsti_a)
        gather_start(i0 + 1, rows_b, gsem_b)
        scatter_start(rows_a, dsti_a, ssem_a, dsem_a)
        gather_drain(rows_b, gsem_b)
        stage(i0 + 1, dsti_b)
        scatter_drain(rows_a, ssem_a, dsem_a)
        scatter_start(rows_b, dsti_b, ssem_b, dsem_b)
        gather_start(i0 + 2, rows_a, gsem_a)
        scatter_drain(rows_b, ssem_b, dsem_b)
        return 0

      lax.fori_loop(0, (CHUNKS_PER_SUPER - 1) // 2, pair_body, 0)

      gather_drain(rows_a, gsem_a)
      stage(CHUNKS_PER_SUPER - 1, dsti_a)
      scatter_start(rows_a, dsti_a, ssem_a, dsem_a)
      scatter_drain(rows_a, ssem_a, dsem_a)
      return 0

    lax.fori_loop(0, NUM_SUPER, super_body, 0)

    plsc.subcore_barrier()

    pltpu.sync_copy(agg_sh.at[pl.ds(r0, ROWS_PER_TILE)],
                    agg_out.at[c, pl.ds(r0, ROWS_PER_TILE)])

    @pl.when(c == 0)
    def _():
      pltpu.sync_copy(deg_sh.at[pl.ds(r0, ROWS_PER_TILE)],
                      deg_out.at[pl.ds(r0, ROWS_PER_TILE)])

  return body(xcat, src, dst, zrow, zdeg, ones)


ROW_BLOCK = 1000
GRID = N_NODES // ROW_BLOCK


def _tc_body(x_ref, aggl_ref, aggr_ref, deg_ref, ws_ref, wnt_ref, wnb_ref,
             b_ref, gamma_ref, beta_ref, out_ref):
  scale = 1.0 / jnp.maximum(deg_ref[:, 0:1], 1.0)
  h = jnp.dot(x_ref[:], ws_ref[:], preferred_element_type=jnp.float32)
  h += jnp.dot(aggl_ref[0] * scale, wnt_ref[:],
               preferred_element_type=jnp.float32)
  h += jnp.dot(aggr_ref[0] * scale, wnb_ref[:],
               preferred_element_type=jnp.float32)
  h += b_ref[:]
  mu = jnp.mean(h, axis=-1, keepdims=True)
  d = h - mu
  var = jnp.mean(d * d, axis=-1, keepdims=True)
  y = d * lax.rsqrt(var + 1e-5) * gamma_ref[:] + beta_ref[:]
  out_ref[:] = jnp.maximum(y, 0.0)


def _tc_dense(x, agg2, deg16, W_self, W_nbr_t, W_nbr_b, b, gamma, beta):
  row_spec = lambda w: pl.BlockSpec((ROW_BLOCK, w), lambda i: (i, 0))
  full_spec = lambda a, b_: pl.BlockSpec((a, b_), lambda i: (0, 0))
  agg_spec = lambda c: pl.BlockSpec((1, ROW_BLOCK, DH), lambda i: (c, i, 0))
  return pl.pallas_call(
      _tc_body,
      grid=(GRID,),
      in_specs=[
          row_spec(D),
          agg_spec(0),
          agg_spec(1),
          row_spec(DEG_W),
          full_spec(D, D),
          full_spec(DH, D),
          full_spec(DH, D),
          full_spec(1, D),
          full_spec(1, D),
          full_spec(1, D),
      ],
      out_specs=row_spec(D),
      out_shape=jax.ShapeDtypeStruct((N_NODES, D), jnp.float32),
  )(x, agg2, agg2, deg16, W_self, W_nbr_t, W_nbr_b, b, gamma, beta)


def kernel(x, edge_index, W_self, W_nbr, b, gamma, beta):
  src = edge_index[0].astype(jnp.int32)
  dst = edge_index[1].astype(jnp.int32)
  xcat = x.reshape(2 * N_NODES, DH)
  zrow = jnp.zeros((ROWS_PER_TILE, DH), jnp.float32)
  zdeg = jnp.zeros((ROWS_PER_TILE, DEG_W), jnp.float32)
  ones = jnp.ones((CHUNK, DEG_W), jnp.float32)

  agg2, deg16 = _sc_segment_sum(xcat, src, dst, zrow, zdeg, ones)

  return _tc_dense(x, agg2, deg16,
                   W_self, W_nbr[:DH], W_nbr[DH:],
                   b.reshape(1, D), gamma.reshape(1, D), beta.reshape(1, D))


# speedup vs baseline: 2.2342x; 1.0017x over previous
import functools

import jax
import jax.numpy as jnp
from jax import lax
from jax.experimental import pallas as pl
from jax.experimental.pallas import tpu as pltpu
from jax.experimental.pallas import tpu_sc as plsc

N_NODES = 10000
N_EDGES = 160000
D = 256
DH = D // 2
NUM_CORES = 2
NUM_SUBCORES = 16
EDGES_PER_TILE = N_EDGES // NUM_SUBCORES
CHUNK = 80
SUPER = 2000
NUM_SUPER = EDGES_PER_TILE // SUPER
CHUNKS_PER_SUPER = SUPER // CHUNK
ROWS_PER_TILE = N_NODES // NUM_SUBCORES
DEG_W = 16


def _sc_segment_sum(xcat, src, dst, zrow, zdeg, ones):
  mesh = plsc.VectorSubcoreMesh(core_axis_name="c", subcore_axis_name="s")

  @functools.partial(
      pl.kernel,
      mesh=mesh,
      compiler_params=pltpu.CompilerParams(use_tc_tiling_on_sc=False),
      out_type=[
          jax.ShapeDtypeStruct((NUM_CORES, N_NODES, DH), jnp.float32),
          jax.ShapeDtypeStruct((N_NODES, DEG_W), jnp.float32),
      ],
      scratch_types=[
          pltpu.VMEM_SHARED((N_NODES, DH), jnp.float32),
          pltpu.VMEM_SHARED((N_NODES, DEG_W), jnp.float32),
          pltpu.VMEM((SUPER,), jnp.int32),
          pltpu.VMEM((SUPER,), jnp.int32),
          pltpu.VMEM((CHUNK, DH), jnp.float32),
          pltpu.VMEM((CHUNK, DH), jnp.float32),
          pltpu.VMEM((CHUNK,), jnp.int32),
          pltpu.VMEM((CHUNK,), jnp.int32),
          pltpu.VMEM((CHUNK, DEG_W), jnp.float32),
          pltpu.SemaphoreType.DMA,
          pltpu.SemaphoreType.DMA,
          pltpu.SemaphoreType.DMA,
          pltpu.SemaphoreType.DMA,
          pltpu.SemaphoreType.DMA,
          pltpu.SemaphoreType.DMA,
      ],
  )
  def body(xcat_hbm, src_hbm, dst_hbm, zrow_hbm, zdeg_hbm, ones_hbm,
           agg_out, deg_out, agg_sh, deg_sh, src_v, dst_v, rows_a, rows_b,
           dsti_a, dsti_b, ones_v, gsem_a, gsem_b, ssem_a, ssem_b, dsem_a,
           dsem_b):
    c = lax.axis_index("c")
    s = lax.axis_index("s")
    e0 = s * EDGES_PER_TILE
    r0 = s * ROWS_PER_TILE

    pltpu.sync_copy(zrow_hbm, agg_sh.at[pl.ds(r0, ROWS_PER_TILE)])

    @pl.when(c == 0)
    def _():
      pltpu.sync_copy(zdeg_hbm, deg_sh.at[pl.ds(r0, ROWS_PER_TILE)])
      pltpu.sync_copy(ones_hbm, ones_v)

    plsc.subcore_barrier()

    cvec = jnp.full((16,), c, dtype=jnp.int32)

    def gather_start(i, rows_buf, sem):
      pltpu.async_copy(
          xcat_hbm.at[src_v.at[pl.ds(i * CHUNK, CHUNK)]], rows_buf, sem)

    def gather_drain(rows_buf, sem):
      pltpu.make_async_copy(xcat_hbm.at[pl.ds(0, CHUNK)], rows_buf, sem).wait()

    def stage(i, dsti_buf):
      for j in range(CHUNK // 16):
        dsti_buf[pl.ds(j * 16, 16)] = dst_v[pl.ds(i * CHUNK + j * 16, 16)]

    def scatter_start(rows_buf, dsti_buf, ssem, dsem):
      pltpu.async_copy(rows_buf, agg_sh.at[dsti_buf], ssem, add=True)

      @pl.when(c == 0)
      def _():
        pltpu.async_copy(ones_v, deg_sh.at[dsti_buf], dsem, add=True)

    def scatter_drain(rows_buf, ssem, dsem):
      pltpu.make_async_copy(rows_buf, agg_sh.at[pl.ds(0, CHUNK)], ssem).wait()

      @pl.when(c == 0)
      def _():
        pltpu.make_async_copy(ones_v, deg_sh.at[pl.ds(0, CHUNK)], dsem).wait()

    def super_body(sc, _):
      pltpu.sync_copy(src_hbm.at[pl.ds(e0 + sc * SUPER, SUPER)], src_v)
      pltpu.sync_copy(dst_hbm.at[pl.ds(e0 + sc * SUPER, SUPER)], dst_v)

      def add_bias(i, _):
        src_v[pl.ds(i * 16, 16)] = src_v[pl.ds(i * 16, 16)] * 2 + cvec
        return 0

      lax.fori_loop(0, SUPER // 16, add_bias, 0)

      gather_start(0, rows_a, gsem_a)

      def pair_body(p, _):
        i0 = 2 * p
        gather_drain(rows_a, gsem_a)
        stage(i0, dsti_a)
        gather_start(i0 + 1, rows_b, gsem_b)
        scatter_start(rows_a, dsti_a, ssem_a, dsem_a)
        gather_drain(rows_b, gsem_b)
        stage(i0 + 1, dsti_b)
        scatter_start(rows_b, dsti_b, ssem_b, dsem_b)
        scatter_drain(rows_a, ssem_a, dsem_a)
        gather_start(i0 + 2, rows_a, gsem_a)
        scatter_drain(rows_b, ssem_b, dsem_b)
        return 0

      lax.fori_loop(0, (CHUNKS_PER_SUPER - 1) // 2, pair_body, 0)

      gather_drain(rows_a, gsem_a)
      stage(CHUNKS_PER_SUPER - 1, dsti_a)
      scatter_start(rows_a, dsti_a, ssem_a, dsem_a)
      scatter_drain(rows_a, ssem_a, dsem_a)
      return 0

    lax.fori_loop(0, NUM_SUPER, super_body, 0)

    plsc.subcore_barrier()

    pltpu.sync_copy(agg_sh.at[pl.ds(r0, ROWS_PER_TILE)],
                    agg_out.at[c, pl.ds(r0, ROWS_PER_TILE)])

    @pl.when(c == 0)
    def _():
      pltpu.sync_copy(deg_sh.at[pl.ds(r0, ROWS_PER_TILE)],
                      deg_out.at[pl.ds(r0, ROWS_PER_TILE)])

  return body(xcat, src, dst, zrow, zdeg, ones)


ROW_BLOCK = 1000
GRID = N_NODES // ROW_BLOCK


def _tc_body(x_ref, aggl_ref, aggr_ref, deg_ref, ws_ref, wnt_ref, wnb_ref,
             b_ref, gamma_ref, beta_ref, out_ref):
  scale = 1.0 / jnp.maximum(deg_ref[:, 0:1], 1.0)
  h = jnp.dot(x_ref[:], ws_ref[:], preferred_element_type=jnp.float32)
  h += jnp.dot(aggl_ref[0] * scale, wnt_ref[:],
               preferred_element_type=jnp.float32)
  h += jnp.dot(aggr_ref[0] * scale, wnb_ref[:],
               preferred_element_type=jnp.float32)
  h += b_ref[:]
  mu = jnp.mean(h, axis=-1, keepdims=True)
  d = h - mu
  var = jnp.mean(d * d, axis=-1, keepdims=True)
  y = d * lax.rsqrt(var + 1e-5) * gamma_ref[:] + beta_ref[:]
  out_ref[:] = jnp.maximum(y, 0.0)


def _tc_dense(x, agg2, deg16, W_self, W_nbr_t, W_nbr_b, b, gamma, beta):
  row_spec = lambda w: pl.BlockSpec((ROW_BLOCK, w), lambda i: (i, 0))
  full_spec = lambda a, b_: pl.BlockSpec((a, b_), lambda i: (0, 0))
  agg_spec = lambda c: pl.BlockSpec((1, ROW_BLOCK, DH), lambda i: (c, i, 0))
  return pl.pallas_call(
      _tc_body,
      grid=(GRID,),
      in_specs=[
          row_spec(D),
          agg_spec(0),
          agg_spec(1),
          row_spec(DEG_W),
          full_spec(D, D),
          full_spec(DH, D),
          full_spec(DH, D),
          full_spec(1, D),
          full_spec(1, D),
          full_spec(1, D),
      ],
      out_specs=row_spec(D),
      out_shape=jax.ShapeDtypeStruct((N_NODES, D), jnp.float32),
  )(x, agg2, agg2, deg16, W_self, W_nbr_t, W_nbr_b, b, gamma, beta)


def kernel(x, edge_index, W_self, W_nbr, b, gamma, beta):
  src = edge_index[0].astype(jnp.int32)
  dst = edge_index[1].astype(jnp.int32)
  xcat = x.reshape(2 * N_NODES, DH)
  zrow = jnp.zeros((ROWS_PER_TILE, DH), jnp.float32)
  zdeg = jnp.zeros((ROWS_PER_TILE, DEG_W), jnp.float32)
  ones = jnp.ones((CHUNK, DEG_W), jnp.float32)

  agg2, deg16 = _sc_segment_sum(xcat, src, dst, zrow, zdeg, ones)

  return _tc_dense(x, agg2, deg16,
                   W_self, W_nbr[:DH], W_nbr[DH:],
                   b.reshape(1, D), gamma.reshape(1, D), beta.reshape(1, D))


# speedup vs baseline: 2.9220x; 1.3078x over previous
import functools

import jax
import jax.numpy as jnp
from jax import lax
from jax.experimental import pallas as pl
from jax.experimental.pallas import tpu as pltpu
from jax.experimental.pallas import tpu_sc as plsc

N_NODES = 10000
N_EDGES = 160000
D = 256
DH = D // 2
NUM_CORES = 2
NUM_SUBCORES = 16
EDGES_PER_TILE = N_EDGES // NUM_SUBCORES
CHUNK = 80
SUPER = 2000
NUM_SUPER = EDGES_PER_TILE // SUPER
CHUNKS_PER_SUPER = SUPER // CHUNK
ROWS_PER_TILE = N_NODES // NUM_SUBCORES
DEG_W = 16


def _sc_segment_sum(xcat, src, dst, zrow, zdeg, ones):
  mesh = plsc.VectorSubcoreMesh(core_axis_name="c", subcore_axis_name="s")

  @functools.partial(
      pl.kernel,
      mesh=mesh,
      compiler_params=pltpu.CompilerParams(use_tc_tiling_on_sc=False),
      out_type=[
          jax.ShapeDtypeStruct((NUM_CORES, N_NODES, DH), jnp.float32),
          jax.ShapeDtypeStruct((N_NODES, DEG_W), jnp.float32),
      ],
      scratch_types=[
          pltpu.VMEM_SHARED((N_NODES, DH), jnp.float32),
          pltpu.VMEM_SHARED((N_NODES, DEG_W), jnp.float32),
          pltpu.VMEM((SUPER,), jnp.int32),
          pltpu.VMEM((SUPER,), jnp.int32),
          pltpu.VMEM((CHUNK, DH), jnp.float32),
          pltpu.VMEM((CHUNK, DH), jnp.float32),
          pltpu.VMEM((CHUNK, DH), jnp.float32),
          pltpu.VMEM((CHUNK,), jnp.int32),
          pltpu.VMEM((CHUNK,), jnp.int32),
          pltpu.VMEM((CHUNK,), jnp.int32),
          pltpu.VMEM((CHUNK, DEG_W), jnp.float32),
          pltpu.SemaphoreType.DMA,
          pltpu.SemaphoreType.DMA,
          pltpu.SemaphoreType.DMA,
          pltpu.SemaphoreType.DMA,
          pltpu.SemaphoreType.DMA,
          pltpu.SemaphoreType.DMA,
          pltpu.SemaphoreType.DMA,
          pltpu.SemaphoreType.DMA,
          pltpu.SemaphoreType.DMA,
      ],
  )
  def body(xcat_hbm, src_hbm, dst_hbm, zrow_hbm, zdeg_hbm, ones_hbm,
           agg_out, deg_out, agg_sh, deg_sh, src_v, dst_v, rows_a, rows_b,
           rows_c, dsti_a, dsti_b, dsti_c, ones_v, gsem_a, gsem_b, gsem_c,
           ssem_a, ssem_b, ssem_c, dsem_a, dsem_b, dsem_c):
    c = lax.axis_index("c")
    s = lax.axis_index("s")
    e0 = s * EDGES_PER_TILE
    r0 = s * ROWS_PER_TILE

    pltpu.sync_copy(zrow_hbm, agg_sh.at[pl.ds(r0, ROWS_PER_TILE)])

    @pl.when(c == 0)
    def _():
      pltpu.sync_copy(zdeg_hbm, deg_sh.at[pl.ds(r0, ROWS_PER_TILE)])
      pltpu.sync_copy(ones_hbm, ones_v)

    plsc.subcore_barrier()

    cvec = jnp.full((16,), c, dtype=jnp.int32)

    def gather_start(i, rows_buf, sem):
      pltpu.async_copy(
          xcat_hbm.at[src_v.at[pl.ds(i * CHUNK, CHUNK)]], rows_buf, sem)

    def gather_drain(rows_buf, sem):
      pltpu.make_async_copy(xcat_hbm.at[pl.ds(0, CHUNK)], rows_buf, sem).wait()

    def stage(i, dsti_buf):
      for j in range(CHUNK // 16):
        dsti_buf[pl.ds(j * 16, 16)] = dst_v[pl.ds(i * CHUNK + j * 16, 16)]

    def scatter_start(rows_buf, dsti_buf, ssem, dsem):
      pltpu.async_copy(rows_buf, agg_sh.at[dsti_buf], ssem, add=True)

      @pl.when(c == 0)
      def _():
        pltpu.async_copy(ones_v, deg_sh.at[dsti_buf], dsem, add=True)

    def scatter_drain(rows_buf, ssem, dsem):
      pltpu.make_async_copy(rows_buf, agg_sh.at[pl.ds(0, CHUNK)], ssem).wait()

      @pl.when(c == 0)
      def _():
        pltpu.make_async_copy(ones_v, deg_sh.at[pl.ds(0, CHUNK)], dsem).wait()

    def super_body(sc, _):
      pltpu.sync_copy(src_hbm.at[pl.ds(e0 + sc * SUPER, SUPER)], src_v)
      pltpu.sync_copy(dst_hbm.at[pl.ds(e0 + sc * SUPER, SUPER)], dst_v)

      def add_bias(i, _):
        src_v[pl.ds(i * 16, 16)] = src_v[pl.ds(i * 16, 16)] * 2 + cvec
        return 0

      lax.fori_loop(0, SUPER // 16, add_bias, 0)

      def step(i, rows_buf, dsti_buf, gsem, ssem, dsem, restart):
        gather_drain(rows_buf, gsem)
        stage(i, dsti_buf)
        scatter_start(rows_buf, dsti_buf, ssem, dsem)
        scatter_drain(rows_buf, ssem, dsem)
        if restart:
          gather_start(i + 3, rows_buf, gsem)

      gather_start(0, rows_a, gsem_a)
      gather_start(1, rows_b, gsem_b)
      gather_start(2, rows_c, gsem_c)

      def triple_body(t, _):
        i0 = 3 * t
        step(i0, rows_a, dsti_a, gsem_a, ssem_a, dsem_a, True)
        step(i0 + 1, rows_b, dsti_b, gsem_b, ssem_b, dsem_b, True)
        step(i0 + 2, rows_c, dsti_c, gsem_c, ssem_c, dsem_c, True)
        return 0

      n_triples = (CHUNKS_PER_SUPER - 4) // 3
      lax.fori_loop(0, n_triples, triple_body, 0)

      i_tail = 3 * n_triples
      step(i_tail, rows_a, dsti_a, gsem_a, ssem_a, dsem_a, True)
      step(i_tail + 1, rows_b, dsti_b, gsem_b, ssem_b, dsem_b, False)
      step(i_tail + 2, rows_c, dsti_c, gsem_c, ssem_c, dsem_c, False)
      step(i_tail + 3, rows_a, dsti_a, gsem_a, ssem_a, dsem_a, False)
      return 0

    lax.fori_loop(0, NUM_SUPER, super_body, 0)

    plsc.subcore_barrier()

    pltpu.sync_copy(agg_sh.at[pl.ds(r0, ROWS_PER_TILE)],
                    agg_out.at[c, pl.ds(r0, ROWS_PER_TILE)])

    @pl.when(c == 0)
    def _():
      pltpu.sync_copy(deg_sh.at[pl.ds(r0, ROWS_PER_TILE)],
                      deg_out.at[pl.ds(r0, ROWS_PER_TILE)])

  return body(xcat, src, dst, zrow, zdeg, ones)


ROW_BLOCK = 1000
GRID = N_NODES // ROW_BLOCK


def _tc_body(x_ref, aggl_ref, aggr_ref, deg_ref, ws_ref, wnt_ref, wnb_ref,
             b_ref, gamma_ref, beta_ref, out_ref):
  scale = 1.0 / jnp.maximum(deg_ref[:, 0:1], 1.0)
  h = jnp.dot(x_ref[:], ws_ref[:], preferred_element_type=jnp.float32)
  h += jnp.dot(aggl_ref[0] * scale, wnt_ref[:],
               preferred_element_type=jnp.float32)
  h += jnp.dot(aggr_ref[0] * scale, wnb_ref[:],
               preferred_element_type=jnp.float32)
  h += b_ref[:]
  mu = jnp.mean(h, axis=-1, keepdims=True)
  d = h - mu
  var = jnp.mean(d * d, axis=-1, keepdims=True)
  y = d * lax.rsqrt(var + 1e-5) * gamma_ref[:] + beta_ref[:]
  out_ref[:] = jnp.maximum(y, 0.0)


def _tc_dense(x, agg2, deg16, W_self, W_nbr_t, W_nbr_b, b, gamma, beta):
  row_spec = lambda w: pl.BlockSpec((ROW_BLOCK, w), lambda i: (i, 0))
  full_spec = lambda a, b_: pl.BlockSpec((a, b_), lambda i: (0, 0))
  agg_spec = lambda c: pl.BlockSpec((1, ROW_BLOCK, DH), lambda i: (c, i, 0))
  return pl.pallas_call(
      _tc_body,
      grid=(GRID,),
      in_specs=[
          row_spec(D),
          agg_spec(0),
          agg_spec(1),
          row_spec(DEG_W),
          full_spec(D, D),
          full_spec(DH, D),
          full_spec(DH, D),
          full_spec(1, D),
          full_spec(1, D),
          full_spec(1, D),
      ],
      out_specs=row_spec(D),
      out_shape=jax.ShapeDtypeStruct((N_NODES, D), jnp.float32),
  )(x, agg2, agg2, deg16, W_self, W_nbr_t, W_nbr_b, b, gamma, beta)


def kernel(x, edge_index, W_self, W_nbr, b, gamma, beta):
  src = edge_index[0].astype(jnp.int32)
  dst = edge_index[1].astype(jnp.int32)
  xcat = x.reshape(2 * N_NODES, DH)
  zrow = jnp.zeros((ROWS_PER_TILE, DH), jnp.float32)
  zdeg = jnp.zeros((ROWS_PER_TILE, DEG_W), jnp.float32)
  ones = jnp.ones((CHUNK, DEG_W), jnp.float32)

  agg2, deg16 = _sc_segment_sum(xcat, src, dst, zrow, zdeg, ones)

  return _tc_dense(x, agg2, deg16,
                   W_self, W_nbr[:DH], W_nbr[DH:],
                   b.reshape(1, D), gamma.reshape(1, D), beta.reshape(1, D))
